# baseline (speedup 1.0000x reference)
import functools
import jax
import jax.numpy as jnp
from jax import lax
from jax.experimental import pallas as pl
from jax.experimental.pallas import tpu as pltpu
from jax.experimental.pallas import tpu_sc as plsc

BINS_ = 32
C_ = 96
N_ = 224 * 224
NT_ = N_ // 128
CBLK = 8
WEIGHT_ = 1.0
NWORK = 32
CPW = C_ // NWORK


_GDN = lax.GatherDimensionNumbers(
    offset_dims=(), collapsed_slice_dims=(0,), start_index_map=(0,))


def _vperm(v, idx):
    return lax.gather(v, idx[:, None], _GDN, slice_sizes=(1,),
                      mode=lax.GatherScatterMode.PROMISE_IN_BOUNDS)


def _vtotal(v, lane):
    for k in (1, 2, 4, 8):
        v = v + _vperm(v, jnp.bitwise_xor(lane, k))
    return v


def _vcumsum(v, lane):
    for k in (1, 2, 4, 8):
        sh = _vperm(v, jnp.maximum(lane - k, 0))
        v = v + jnp.where(lane >= k, sh, 0.0)
    return v


def _sc_fine(style_hbm, inp_hbm, mask_hbm, fine_hbm,
             st_v, idx0, idx1, idx2, tidx0, tidx1, tidx2, rem_v,
             xr0, xr1, xr2, mr0, mr1, mr2, fine_v, sem):
    i32 = jnp.int32
    f32 = jnp.float32
    wid = lax.axis_index("s") * 2 + lax.axis_index("c")
    lane = lax.iota(i32, 16)
    idx_refs = (idx0, idx1, idx2)
    tidx_refs = (tidx0, tidx1, tidx2)
    xr_refs = (xr0, xr1, xr2)
    mr_refs = (mr0, mr1, mr2)

    copies = []
    for k in range(CPW):
        ch = wid * CPW + k
        pltpu.sync_copy(style_hbm.at[ch], st_v)
        lo = st_v[pl.ds(0, 16)]
        hi = st_v[pl.ds(16, 16)]
        rs = _vtotal(lo + hi, lane)
        sc = float(N_) / rs
        shlo = lo * sc
        shhi = hi * sc
        cslo = _vcumsum(shlo, lane)
        cshi = _vcumsum(shhi, lane) + _vtotal(shlo, lane)
        klo = jnp.clip(cslo.astype(i32), 0, N_)
        khi = jnp.clip(cshi.astype(i32), 0, N_)
        tlo = jnp.minimum(jnp.right_shift(klo, 7), NT_ - 1)
        thi = jnp.minimum(jnp.right_shift(khi, 7), NT_ - 1)
        idx_refs[k][pl.ds(0, 16)] = ch * NT_ + tlo
        idx_refs[k][pl.ds(16, 16)] = ch * NT_ + thi
        tidx_refs[k][pl.ds(0, 16)] = tlo
        tidx_refs[k][pl.ds(16, 16)] = thi
        rem_v[pl.ds(k * BINS_, 16)] = klo - tlo * 128
        rem_v[pl.ds(k * BINS_ + 16, 16)] = khi - thi * 128
        copies.append(pltpu.async_copy(inp_hbm.at[idx_refs[k]], xr_refs[k], sem))
        copies.append(pltpu.async_copy(mask_hbm.at[tidx_refs[k]], mr_refs[k], sem))
    for cp in copies:
        cp.wait()

    for k in range(CPW):
        ch = wid * CPW + k
        xrows = xr_refs[k]
        mrows = mr_refs[k]

        def body(b, carry):
            f0, f1 = carry
            rb = rem_v[pl.ds(k * BINS_ + b, 1)][0]
            acc = jnp.zeros((16,), f32)
            for j in range(8):
                xv = xrows[b, pl.ds(j * 16, 16)]
                mv = mrows[b, pl.ds(j * 16, 16)]
                pv = lane + (j * 16)
                acc = acc + jnp.where(pv >= rb, xv * mv, 0.0)
            s = _vtotal(acc, lane)
            f0 = jnp.where(lane == b, s, f0)
            f1 = jnp.where(lane == b - 16, s, f1)
            return f0, f1

        z16 = jnp.zeros((16,), f32)
        f0, f1 = lax.fori_loop(0, BINS_, body, (z16, z16))
        fine_v[pl.ds(0, 16)] = f0
        fine_v[pl.ds(16, 16)] = f1
        pltpu.sync_copy(fine_v, fine_hbm.at[ch])


def _stream_kernel(inp_ref, mask_ref, style_ref, out_ref, scr_ref, acc_ref):
    i = pl.program_id(0)
    nblk = pl.num_programs(0)
    f32 = jnp.float32

    st = style_ref[...]
    rs = jnp.sum(st, axis=1, keepdims=True)
    sh = st * (float(N_) / rs)
    r = lax.broadcasted_iota(jnp.int32, (BINS_, BINS_), 0)
    c = lax.broadcasted_iota(jnp.int32, (BINS_, BINS_), 1)
    tri_up = jnp.where(r <= c, 1.0, 0.0).astype(f32)
    cdf = jnp.dot(sh, tri_up, preferred_element_type=f32)
    Kf = jnp.clip(jnp.floor(cdf), 0.0, float(N_))
    Ki = Kf.astype(jnp.int32)
    t = jnp.minimum(Ki // 128, NT_ - 1)
    q_part = jnp.sum(float(N_) - Kf, axis=0, keepdims=True)

    x = inp_ref[...] * mask_ref[...]
    x2d = jnp.reshape(x, (CBLK * NT_, 128))
    ones_col = jnp.ones((128, 1), f32)
    tsall = jnp.dot(x2d, ones_col, preferred_element_type=f32)
    ssall = jnp.dot(x2d * x2d, ones_col, preferred_element_type=f32)
    ss_part = jnp.sum(ssall)
    stot_part = jnp.sum(tsall)

    i392 = lax.broadcasted_iota(jnp.int32, (NT_, 1), 0)
    t_row_acc = jnp.zeros((1, BINS_), f32)
    for cl in range(CBLK):
        tsc = tsall[cl * NT_:(cl + 1) * NT_]
        t_row = t[cl:cl + 1, :]
        cmpgt = jnp.where(i392 > t_row, 1.0, 0.0).astype(f32)
        coarse = lax.dot_general(tsc, cmpgt, (((0,), (0,)), ((), ())),
                                 preferred_element_type=f32)
        t_row_acc = t_row_acc + coarse

    @pl.when(i == 0)
    def _():
        scr_ref[0:1, :] = t_row_acc
        scr_ref[1:2, :] = q_part
        scr_ref[2:3, :] = cdf[0:1, :]
        acc_ref[0] = ss_part
        acc_ref[1] = jnp.sum(mask_ref[...])
        acc_ref[2] = jnp.min(x[0])
        acc_ref[3] = jnp.max(x[0])
        acc_ref[4] = stot_part
        acc_ref[5] = jnp.sum(cdf[1:2, 0:1])

    @pl.when(i > 0)
    def _():
        scr_ref[0:1, :] = scr_ref[0:1, :] + t_row_acc
        scr_ref[1:2, :] = scr_ref[1:2, :] + q_part
        acc_ref[0] = acc_ref[0] + ss_part
        acc_ref[4] = acc_ref[4] + stot_part

    @pl.when(i == nblk - 1)
    def _():
        out_ref[0:1, 0:BINS_] = scr_ref[0:1, :]
        out_ref[1:2, 0:BINS_] = scr_ref[1:2, :]
        out_ref[2:3, 0:BINS_] = scr_ref[2:3, :]
        out_ref[3:4, 0:1] = jnp.reshape(acc_ref[0], (1, 1))
        out_ref[3:4, 1:2] = jnp.reshape(acc_ref[1], (1, 1))
        out_ref[3:4, 2:3] = jnp.reshape(acc_ref[2], (1, 1))
        out_ref[3:4, 3:4] = jnp.reshape(acc_ref[3], (1, 1))
        out_ref[3:4, 4:5] = jnp.reshape(acc_ref[4], (1, 1))
        out_ref[3:4, 5:6] = jnp.reshape(acc_ref[5], (1, 1))


def _combine_kernel(p_ref, fine_ref, out_ref):
    f32 = jnp.float32
    t_row = p_ref[0:1, 0:BINS_] + jnp.sum(fine_ref[...], axis=0, keepdims=True)
    q_row = p_ref[1:2, 0:BINS_]
    cdf0_row = p_ref[2:3, 0:BINS_]
    ss = p_ref[3:4, 0:1]
    msum = p_ref[3:4, 1:2]
    mn0 = p_ref[3:4, 2:3]
    mx0 = p_ref[3:4, 3:4]
    stot = p_ref[3:4, 4:5]
    cdf1_0 = p_ref[3:4, 5:6]
    step0 = (mx0 - mn0) / BINS_

    m1 = lax.broadcasted_iota(jnp.int32, (64, 1), 0).astype(f32) + 1.0
    idx0 = jnp.sum(jnp.where(cdf0_row < m1, 1.0, 0.0),
                   axis=1, keepdims=True)
    jrow = lax.broadcasted_iota(jnp.int32, (1, 64), 1).astype(f32)
    eq = jnp.where(idx0 == jrow, 1.0, 0.0)
    z1 = jnp.zeros((1, 1), f32)
    cdfp_ext = jnp.concatenate(
        [z1, cdf0_row[:, 0:31], jnp.zeros((1, 32), f32)], axis=1)
    cdf_ext = jnp.concatenate(
        [cdf0_row, cdf1_0, jnp.zeros((1, 31), f32)], axis=1)
    cdfp_sel = jnp.sum(eq * cdfp_ext, axis=1, keepdims=True)
    cdf_sel = jnp.sum(eq * cdf_ext, axis=1, keepdims=True)
    ratio = jnp.clip((m1 - cdfp_sel) / (1e-8 + cdf_sel), 0.0, 1.0)
    lut = mn0 + (ratio + idx0) * step0

    dlut = lut[1:33] - lut[0:32]
    lutsq = lut * lut
    dlut2 = lutsq[1:33] - lutsq[0:32]
    lut0 = lut[0:1, 0:1]

    cross = jnp.dot(t_row, dlut, preferred_element_type=f32)
    lut2t = jnp.dot(q_row, dlut2, preferred_element_type=f32)
    total = float(C_) * float(N_)
    loss_sum = (ss - 2.0 * (lut0 * stot + cross)
                + (total * lut0 * lut0 + lut2t))
    scale = (float(C_) / (total * total)) * WEIGHT_
    out_ref[...] = (loss_sum * scale) * msum


def kernel(input, mask_tight, mask_rough, style_his):
    inp3 = input.reshape(C_, NT_, 128)
    inp2d = input.reshape(C_ * NT_, 128)
    mask3 = mask_tight.reshape(1, NT_, 128)
    mask2d = mask_tight.reshape(NT_, 128)

    sc_call = functools.partial(
        pl.kernel,
        mesh=plsc.VectorSubcoreMesh(core_axis_name="c", subcore_axis_name="s"),
        out_type=jax.ShapeDtypeStruct((C_, BINS_), jnp.float32),
        scratch_types=[
            pltpu.VMEM((BINS_,), jnp.float32),
            pltpu.VMEM((BINS_,), jnp.int32),
            pltpu.VMEM((BINS_,), jnp.int32),
            pltpu.VMEM((BINS_,), jnp.int32),
            pltpu.VMEM((BINS_,), jnp.int32),
            pltpu.VMEM((BINS_,), jnp.int32),
            pltpu.VMEM((BINS_,), jnp.int32),
            pltpu.VMEM((CPW * BINS_,), jnp.int32),
            pltpu.VMEM((BINS_, 128), jnp.float32),
            pltpu.VMEM((BINS_, 128), jnp.float32),
            pltpu.VMEM((BINS_, 128), jnp.float32),
            pltpu.VMEM((BINS_, 128), jnp.float32),
            pltpu.VMEM((BINS_, 128), jnp.float32),
            pltpu.VMEM((BINS_, 128), jnp.float32),
            pltpu.VMEM((BINS_,), jnp.float32),
            pltpu.SemaphoreType.DMA,
        ],
    )(_sc_fine)
    fine = sc_call(style_his, inp2d, mask2d)

    partials = pl.pallas_call(
        _stream_kernel,
        grid=(C_ // CBLK,),
        in_specs=[
            pl.BlockSpec((CBLK, NT_, 128), lambda i: (i, 0, 0)),
            pl.BlockSpec((1, NT_, 128), lambda i: (0, 0, 0)),
            pl.BlockSpec((CBLK, BINS_), lambda i: (i, 0)),
        ],
        out_specs=pl.BlockSpec((8, 64), lambda i: (0, 0)),
        out_shape=jax.ShapeDtypeStruct((8, 64), jnp.float32),
        scratch_shapes=[
            pltpu.VMEM((8, BINS_), jnp.float32),
            pltpu.SMEM((8,), jnp.float32),
        ],
    )(inp3, mask3, style_his)

    out = pl.pallas_call(
        _combine_kernel,
        grid=(1,),
        in_specs=[
            pl.BlockSpec((8, 64), lambda i: (0, 0)),
            pl.BlockSpec((C_, BINS_), lambda i: (0, 0)),
        ],
        out_specs=pl.BlockSpec((1, 1), lambda i: (0, 0)),
        out_shape=jax.ShapeDtypeStruct((1, 1), jnp.float32),
    )(partials, fine)

    return out[0, 0]


# speedup vs baseline: 1.0184x; 1.0184x over previous
import functools
import jax
import jax.numpy as jnp
from jax import lax
from jax.experimental import pallas as pl
from jax.experimental.pallas import tpu as pltpu
from jax.experimental.pallas import tpu_sc as plsc

BINS_ = 32
C_ = 96
N_ = 224 * 224
NT_ = N_ // 128
CBLK = 8
WEIGHT_ = 1.0
NWORK = 32
CPW = C_ // NWORK


_GDN = lax.GatherDimensionNumbers(
    offset_dims=(), collapsed_slice_dims=(0,), start_index_map=(0,))


def _vperm(v, idx):
    return lax.gather(v, idx[:, None], _GDN, slice_sizes=(1,),
                      mode=lax.GatherScatterMode.PROMISE_IN_BOUNDS)


def _vtotal(v, lane):
    for k in (1, 2, 4, 8):
        v = v + _vperm(v, jnp.bitwise_xor(lane, k))
    return v


def _vcumsum(v, lane):
    for k in (1, 2, 4, 8):
        sh = _vperm(v, jnp.maximum(lane - k, 0))
        v = v + jnp.where(lane >= k, sh, 0.0)
    return v


def _sc_fine(style_hbm, inp_hbm, mask_hbm, fine_hbm,
             st_v, idx0, idx1, idx2, tidx0, tidx1, tidx2, rem_v,
             xr0, xr1, xr2, mr0, mr1, mr2, fine_v, sem):
    i32 = jnp.int32
    f32 = jnp.float32
    wid = lax.axis_index("s") * 2 + lax.axis_index("c")
    lane = lax.iota(i32, 16)
    idx_refs = (idx0, idx1, idx2)
    tidx_refs = (tidx0, tidx1, tidx2)
    xr_refs = (xr0, xr1, xr2)
    mr_refs = (mr0, mr1, mr2)

    copies = []
    for k in range(CPW):
        ch = wid * CPW + k
        pltpu.sync_copy(style_hbm.at[ch], st_v)
        lo = st_v[pl.ds(0, 16)]
        hi = st_v[pl.ds(16, 16)]
        rs = _vtotal(lo + hi, lane)
        sc = float(N_) / rs
        shlo = lo * sc
        shhi = hi * sc
        cslo = _vcumsum(shlo, lane)
        cshi = _vcumsum(shhi, lane) + _vtotal(shlo, lane)
        klo = jnp.clip(cslo.astype(i32), 0, N_)
        khi = jnp.clip(cshi.astype(i32), 0, N_)
        tlo = jnp.minimum(jnp.right_shift(klo, 7), NT_ - 1)
        thi = jnp.minimum(jnp.right_shift(khi, 7), NT_ - 1)
        idx_refs[k][pl.ds(0, 16)] = ch * NT_ + tlo
        idx_refs[k][pl.ds(16, 16)] = ch * NT_ + thi
        tidx_refs[k][pl.ds(0, 16)] = tlo
        tidx_refs[k][pl.ds(16, 16)] = thi
        rem_v[pl.ds(k * BINS_, 16)] = klo - tlo * 128
        rem_v[pl.ds(k * BINS_ + 16, 16)] = khi - thi * 128
        copies.append(pltpu.async_copy(inp_hbm.at[idx_refs[k]], xr_refs[k], sem))
        copies.append(pltpu.async_copy(mask_hbm.at[tidx_refs[k]], mr_refs[k], sem))
    for cp in copies:
        cp.wait()

    for k in range(CPW):
        ch = wid * CPW + k
        xrows = xr_refs[k]
        mrows = mr_refs[k]

        def body(b, carry):
            f0, f1 = carry
            rb = rem_v[pl.ds(k * BINS_ + b, 1)][0]
            acc = jnp.zeros((16,), f32)
            for j in range(8):
                xv = xrows[b, pl.ds(j * 16, 16)]
                mv = mrows[b, pl.ds(j * 16, 16)]
                pv = lane + (j * 16)
                acc = acc + jnp.where(pv >= rb, xv * mv, 0.0)
            s = _vtotal(acc, lane)
            f0 = jnp.where(lane == b, s, f0)
            f1 = jnp.where(lane == b - 16, s, f1)
            return f0, f1

        z16 = jnp.zeros((16,), f32)
        f0, f1 = lax.fori_loop(0, BINS_, body, (z16, z16))
        fine_v[pl.ds(0, 16)] = f0
        fine_v[pl.ds(16, 16)] = f1
        pltpu.sync_copy(fine_v, fine_hbm.at[ch])


def _stream_kernel(inp_ref, mask_ref, style_ref, fine_ref, out_ref,
                   scr_ref, acc_ref):
    i = pl.program_id(0)
    nblk = pl.num_programs(0)
    f32 = jnp.float32

    st = style_ref[...]
    rs = jnp.sum(st, axis=1, keepdims=True)
    sh = st * (float(N_) / rs)
    r = lax.broadcasted_iota(jnp.int32, (BINS_, BINS_), 0)
    c = lax.broadcasted_iota(jnp.int32, (BINS_, BINS_), 1)
    tri_up = jnp.where(r <= c, 1.0, 0.0).astype(f32)
    cdf = jnp.dot(sh, tri_up, preferred_element_type=f32)
    Kf = jnp.clip(jnp.floor(cdf), 0.0, float(N_))
    Ki = Kf.astype(jnp.int32)
    t = jnp.minimum(Ki // 128, NT_ - 1)
    q_part = jnp.sum(float(N_) - Kf, axis=0, keepdims=True)

    x = inp_ref[...] * mask_ref[...]
    x2d = jnp.reshape(x, (CBLK * NT_, 128))
    ones_col = jnp.ones((128, 1), f32)
    tsall = jnp.dot(x2d, ones_col, preferred_element_type=f32)
    ssall = jnp.dot(x2d * x2d, ones_col, preferred_element_type=f32)
    ss_part = jnp.sum(ssall)
    stot_part = jnp.sum(tsall)

    i392 = lax.broadcasted_iota(jnp.int32, (NT_, 1), 0)
    t_row_acc = jnp.zeros((1, BINS_), f32)
    for cl in range(CBLK):
        tsc = tsall[cl * NT_:(cl + 1) * NT_]
        t_row = t[cl:cl + 1, :]
        cmpgt = jnp.where(i392 > t_row, 1.0, 0.0).astype(f32)
        coarse = lax.dot_general(tsc, cmpgt, (((0,), (0,)), ((), ())),
                                 preferred_element_type=f32)
        t_row_acc = t_row_acc + coarse

    @pl.when(i == 0)
    def _():
        scr_ref[0:1, :] = t_row_acc
        scr_ref[1:2, :] = q_part
        scr_ref[2:3, :] = cdf[0:1, :]
        acc_ref[0] = ss_part
        acc_ref[1] = jnp.sum(mask_ref[...])
        acc_ref[2] = jnp.min(x[0])
        acc_ref[3] = jnp.max(x[0])
        acc_ref[4] = stot_part
        acc_ref[5] = jnp.sum(cdf[1:2, 0:1])

    @pl.when(i > 0)
    def _():
        scr_ref[0:1, :] = scr_ref[0:1, :] + t_row_acc
        scr_ref[1:2, :] = scr_ref[1:2, :] + q_part
        acc_ref[0] = acc_ref[0] + ss_part
        acc_ref[4] = acc_ref[4] + stot_part

    @pl.when(i == nblk - 1)
    def _():
        mn0 = acc_ref[2]
        mx0 = acc_ref[3]
        step0 = (mx0 - mn0) / BINS_
        t_row2 = scr_ref[0:1, :] + jnp.sum(fine_ref[...], axis=0, keepdims=True)
        cdf0_row = scr_ref[2:3, :]
        cdf1_0 = jnp.reshape(acc_ref[5], (1, 1))
        m1 = lax.broadcasted_iota(jnp.int32, (64, 1), 0).astype(f32) + 1.0
        idx0 = jnp.sum(jnp.where(cdf0_row < m1, 1.0, 0.0),
                       axis=1, keepdims=True)
        jrow = lax.broadcasted_iota(jnp.int32, (1, 64), 1).astype(f32)
        eq = jnp.where(idx0 == jrow, 1.0, 0.0)
        z1 = jnp.zeros((1, 1), f32)
        cdfp_ext = jnp.concatenate(
            [z1, cdf0_row[:, 0:31], jnp.zeros((1, 32), f32)], axis=1)
        cdf_ext = jnp.concatenate(
            [cdf0_row, cdf1_0, jnp.zeros((1, 31), f32)], axis=1)
        cdfp_sel = jnp.sum(eq * cdfp_ext, axis=1, keepdims=True)
        cdf_sel = jnp.sum(eq * cdf_ext, axis=1, keepdims=True)
        ratio = jnp.clip((m1 - cdfp_sel) / (1e-8 + cdf_sel), 0.0, 1.0)
        lut = mn0 + (ratio + idx0) * step0

        dlut = lut[1:33] - lut[0:32]
        lutsq = lut * lut
        dlut2 = lutsq[1:33] - lutsq[0:32]
        lut0 = lut[0:1, 0:1]

        cross = jnp.dot(t_row2, dlut, preferred_element_type=f32)
        lut2t = jnp.dot(scr_ref[1:2, :], dlut2,
                        preferred_element_type=f32)
        total = float(C_) * float(N_)
        loss_sum = (acc_ref[0] - 2.0 * (lut0 * acc_ref[4] + cross)
                    + (total * lut0 * lut0 + lut2t))
        scale = (float(C_) / (total * total)) * WEIGHT_
        out_ref[...] = (loss_sum * scale) * acc_ref[1]


def kernel(input, mask_tight, mask_rough, style_his):
    inp3 = input.reshape(C_, NT_, 128)
    inp2d = input.reshape(C_ * NT_, 128)
    mask3 = mask_tight.reshape(1, NT_, 128)
    mask2d = mask_tight.reshape(NT_, 128)

    sc_call = functools.partial(
        pl.kernel,
        mesh=plsc.VectorSubcoreMesh(core_axis_name="c", subcore_axis_name="s"),
        out_type=jax.ShapeDtypeStruct((C_, BINS_), jnp.float32),
        scratch_types=[
            pltpu.VMEM((BINS_,), jnp.float32),
            pltpu.VMEM((BINS_,), jnp.int32),
            pltpu.VMEM((BINS_,), jnp.int32),
            pltpu.VMEM((BINS_,), jnp.int32),
            pltpu.VMEM((BINS_,), jnp.int32),
            pltpu.VMEM((BINS_,), jnp.int32),
            pltpu.VMEM((BINS_,), jnp.int32),
            pltpu.VMEM((CPW * BINS_,), jnp.int32),
            pltpu.VMEM((BINS_, 128), jnp.float32),
            pltpu.VMEM((BINS_, 128), jnp.float32),
            pltpu.VMEM((BINS_, 128), jnp.float32),
            pltpu.VMEM((BINS_, 128), jnp.float32),
            pltpu.VMEM((BINS_, 128), jnp.float32),
            pltpu.VMEM((BINS_, 128), jnp.float32),
            pltpu.VMEM((BINS_,), jnp.float32),
            pltpu.SemaphoreType.DMA,
        ],
    )(_sc_fine)
    fine = sc_call(style_his, inp2d, mask2d)

    out = pl.pallas_call(
        _stream_kernel,
        grid=(C_ // CBLK,),
        in_specs=[
            pl.BlockSpec((CBLK, NT_, 128), lambda i: (i, 0, 0)),
            pl.BlockSpec((1, NT_, 128), lambda i: (0, 0, 0)),
            pl.BlockSpec((CBLK, BINS_), lambda i: (i, 0)),
            pl.BlockSpec((C_, BINS_), lambda i: (0, 0)),
        ],
        out_specs=pl.BlockSpec((1, 1), lambda i: (0, 0)),
        out_shape=jax.ShapeDtypeStruct((1, 1), jnp.float32),
        scratch_shapes=[
            pltpu.VMEM((8, BINS_), jnp.float32),
            pltpu.SMEM((8,), jnp.float32),
        ],
    )(inp3, mask3, style_his, fine)

    return out[0, 0]


# speedup vs baseline: 1.0191x; 1.0006x over previous
import functools
import jax
import jax.numpy as jnp
from jax import lax
from jax.experimental import pallas as pl
from jax.experimental.pallas import tpu as pltpu
from jax.experimental.pallas import tpu_sc as plsc

BINS_ = 32
C_ = 96
N_ = 224 * 224
NT_ = N_ // 128
CBLK = 8
WEIGHT_ = 1.0
NWORK = 32
CPW = C_ // NWORK


_GDN = lax.GatherDimensionNumbers(
    offset_dims=(), collapsed_slice_dims=(0,), start_index_map=(0,))


def _vperm(v, idx):
    return lax.gather(v, idx[:, None], _GDN, slice_sizes=(1,),
                      mode=lax.GatherScatterMode.PROMISE_IN_BOUNDS)


def _vtotal(v, lane):
    for k in (1, 2, 4, 8):
        v = v + _vperm(v, jnp.bitwise_xor(lane, k))
    return v


def _vcumsum(v, lane):
    for k in (1, 2, 4, 8):
        sh = _vperm(v, jnp.maximum(lane - k, 0))
        v = v + jnp.where(lane >= k, sh, 0.0)
    return v


def _sc_fine(style_hbm, inp_hbm, mask_hbm, fine_hbm,
             st_v, idx0, idx1, idx2, tidx0, tidx1, tidx2, rem_v,
             xr0, xr1, xr2, mr0, mr1, mr2, fine_v, sem):
    i32 = jnp.int32
    f32 = jnp.float32
    wid = lax.axis_index("s") * 2 + lax.axis_index("c")
    lane = lax.iota(i32, 16)
    idx_refs = (idx0, idx1, idx2)
    tidx_refs = (tidx0, tidx1, tidx2)
    xr_refs = (xr0, xr1, xr2)
    mr_refs = (mr0, mr1, mr2)

    copies = []
    for k in range(CPW):
        ch = wid * CPW + k
        pltpu.sync_copy(style_hbm.at[ch], st_v)
        lo = st_v[pl.ds(0, 16)]
        hi = st_v[pl.ds(16, 16)]
        rs = _vtotal(lo + hi, lane)
        sc = float(N_) / rs
        shlo = lo * sc
        shhi = hi * sc
        cslo = _vcumsum(shlo, lane)
        cshi = _vcumsum(shhi, lane) + _vtotal(shlo, lane)
        klo = jnp.clip(cslo.astype(i32), 0, N_)
        khi = jnp.clip(cshi.astype(i32), 0, N_)
        tlo = jnp.minimum(jnp.right_shift(klo, 7), NT_ - 1)
        thi = jnp.minimum(jnp.right_shift(khi, 7), NT_ - 1)
        idx_refs[k][pl.ds(0, 16)] = ch * NT_ + tlo
        idx_refs[k][pl.ds(16, 16)] = ch * NT_ + thi
        tidx_refs[k][pl.ds(0, 16)] = tlo
        tidx_refs[k][pl.ds(16, 16)] = thi
        rem_v[pl.ds(k * BINS_, 16)] = klo - tlo * 128
        rem_v[pl.ds(k * BINS_ + 16, 16)] = khi - thi * 128
        copies.append(pltpu.async_copy(inp_hbm.at[idx_refs[k]], xr_refs[k], sem))
        copies.append(pltpu.async_copy(mask_hbm.at[tidx_refs[k]], mr_refs[k], sem))
    for cp in copies:
        cp.wait()

    for k in range(CPW):
        ch = wid * CPW + k
        xrows = xr_refs[k]
        mrows = mr_refs[k]

        def body(b, carry):
            f0, f1 = carry
            rb = rem_v[pl.ds(k * BINS_ + b, 1)][0]

            def jbody(j, acc):
                xv = xrows[b, pl.ds(j * 16, 16)]
                mv = mrows[b, pl.ds(j * 16, 16)]
                pv = lane + j * 16
                return acc + jnp.where(pv >= rb, xv * mv, 0.0)

            acc = lax.fori_loop(0, 8, jbody, jnp.zeros((16,), f32))
            s = _vtotal(acc, lane)
            f0 = jnp.where(lane == b, s, f0)
            f1 = jnp.where(lane == b - 16, s, f1)
            return f0, f1

        z16 = jnp.zeros((16,), f32)
        f0, f1 = lax.fori_loop(0, BINS_, body, (z16, z16))
        fine_v[pl.ds(0, 16)] = f0
        fine_v[pl.ds(16, 16)] = f1
        pltpu.sync_copy(fine_v, fine_hbm.at[ch])


def _stream_kernel(inp_ref, mask_ref, style_ref, fine_ref, out_ref,
                   scr_ref, acc_ref):
    i = pl.program_id(0)
    nblk = pl.num_programs(0)
    f32 = jnp.float32

    st = style_ref[...]
    rs = jnp.sum(st, axis=1, keepdims=True)
    sh = st * (float(N_) / rs)
    r = lax.broadcasted_iota(jnp.int32, (BINS_, BINS_), 0)
    c = lax.broadcasted_iota(jnp.int32, (BINS_, BINS_), 1)
    tri_up = jnp.where(r <= c, 1.0, 0.0).astype(f32)
    cdf = jnp.dot(sh, tri_up, preferred_element_type=f32)
    Kf = jnp.clip(jnp.floor(cdf), 0.0, float(N_))
    Ki = Kf.astype(jnp.int32)
    t = jnp.minimum(Ki // 128, NT_ - 1)
    q_part = jnp.sum(float(N_) - Kf, axis=0, keepdims=True)

    x = inp_ref[...] * mask_ref[...]
    x2d = jnp.reshape(x, (CBLK * NT_, 128))
    ones_col = jnp.ones((128, 1), f32)
    tsall = jnp.dot(x2d, ones_col, preferred_element_type=f32)
    ssall = jnp.dot(x2d * x2d, ones_col, preferred_element_type=f32)
    ss_part = jnp.sum(ssall)
    stot_part = jnp.sum(tsall)

    i392 = lax.broadcasted_iota(jnp.int32, (NT_, 1), 0)
    t_row_acc = jnp.zeros((1, BINS_), f32)
    for cl in range(CBLK):
        tsc = tsall[cl * NT_:(cl + 1) * NT_]
        t_row = t[cl:cl + 1, :]
        cmpgt = jnp.where(i392 > t_row, 1.0, 0.0).astype(f32)
        coarse = lax.dot_general(tsc, cmpgt, (((0,), (0,)), ((), ())),
                                 preferred_element_type=f32)
        t_row_acc = t_row_acc + coarse

    @pl.when(i == 0)
    def _():
        scr_ref[0:1, :] = t_row_acc
        scr_ref[1:2, :] = q_part
        scr_ref[2:3, :] = cdf[0:1, :]
        acc_ref[0] = ss_part
        acc_ref[1] = jnp.sum(mask_ref[...])
        acc_ref[2] = jnp.min(x[0])
        acc_ref[3] = jnp.max(x[0])
        acc_ref[4] = stot_part
        acc_ref[5] = jnp.sum(cdf[1:2, 0:1])

    @pl.when(i > 0)
    def _():
        scr_ref[0:1, :] = scr_ref[0:1, :] + t_row_acc
        scr_ref[1:2, :] = scr_ref[1:2, :] + q_part
        acc_ref[0] = acc_ref[0] + ss_part
        acc_ref[4] = acc_ref[4] + stot_part

    @pl.when(i == nblk - 1)
    def _():
        mn0 = acc_ref[2]
        mx0 = acc_ref[3]
        step0 = (mx0 - mn0) / BINS_
        t_row2 = scr_ref[0:1, :] + jnp.sum(fine_ref[...], axis=0, keepdims=True)
        cdf0_row = scr_ref[2:3, :]
        cdf1_0 = jnp.reshape(acc_ref[5], (1, 1))
        m1 = lax.broadcasted_iota(jnp.int32, (64, 1), 0).astype(f32) + 1.0
        idx0 = jnp.sum(jnp.where(cdf0_row < m1, 1.0, 0.0),
                       axis=1, keepdims=True)
        jrow = lax.broadcasted_iota(jnp.int32, (1, 64), 1).astype(f32)
        eq = jnp.where(idx0 == jrow, 1.0, 0.0)
        z1 = jnp.zeros((1, 1), f32)
        cdfp_ext = jnp.concatenate(
            [z1, cdf0_row[:, 0:31], jnp.zeros((1, 32), f32)], axis=1)
        cdf_ext = jnp.concatenate(
            [cdf0_row, cdf1_0, jnp.zeros((1, 31), f32)], axis=1)
        cdfp_sel = jnp.sum(eq * cdfp_ext, axis=1, keepdims=True)
        cdf_sel = jnp.sum(eq * cdf_ext, axis=1, keepdims=True)
        ratio = jnp.clip((m1 - cdfp_sel) / (1e-8 + cdf_sel), 0.0, 1.0)
        lut = mn0 + (ratio + idx0) * step0

        dlut = lut[1:33] - lut[0:32]
        lutsq = lut * lut
        dlut2 = lutsq[1:33] - lutsq[0:32]
        lut0 = lut[0:1, 0:1]

        cross = jnp.dot(t_row2, dlut, preferred_element_type=f32)
        lut2t = jnp.dot(scr_ref[1:2, :], dlut2,
                        preferred_element_type=f32)
        total = float(C_) * float(N_)
        loss_sum = (acc_ref[0] - 2.0 * (lut0 * acc_ref[4] + cross)
                    + (total * lut0 * lut0 + lut2t))
        scale = (float(C_) / (total * total)) * WEIGHT_
        out_ref[...] = (loss_sum * scale) * acc_ref[1]


def kernel(input, mask_tight, mask_rough, style_his):
    inp3 = input.reshape(C_, NT_, 128)
    inp2d = input.reshape(C_ * NT_, 128)
    mask3 = mask_tight.reshape(1, NT_, 128)
    mask2d = mask_tight.reshape(NT_, 128)

    sc_call = functools.partial(
        pl.kernel,
        mesh=plsc.VectorSubcoreMesh(core_axis_name="c", subcore_axis_name="s"),
        out_type=jax.ShapeDtypeStruct((C_, BINS_), jnp.float32),
        scratch_types=[
            pltpu.VMEM((BINS_,), jnp.float32),
            pltpu.VMEM((BINS_,), jnp.int32),
            pltpu.VMEM((BINS_,), jnp.int32),
            pltpu.VMEM((BINS_,), jnp.int32),
            pltpu.VMEM((BINS_,), jnp.int32),
            pltpu.VMEM((BINS_,), jnp.int32),
            pltpu.VMEM((BINS_,), jnp.int32),
            pltpu.VMEM((CPW * BINS_,), jnp.int32),
            pltpu.VMEM((BINS_, 128), jnp.float32),
            pltpu.VMEM((BINS_, 128), jnp.float32),
            pltpu.VMEM((BINS_, 128), jnp.float32),
            pltpu.VMEM((BINS_, 128), jnp.float32),
            pltpu.VMEM((BINS_, 128), jnp.float32),
            pltpu.VMEM((BINS_, 128), jnp.float32),
            pltpu.VMEM((BINS_,), jnp.float32),
            pltpu.SemaphoreType.DMA,
        ],
    )(_sc_fine)
    fine = sc_call(style_his, inp2d, mask2d)

    out = pl.pallas_call(
        _stream_kernel,
        grid=(C_ // CBLK,),
        in_specs=[
            pl.BlockSpec((CBLK, NT_, 128), lambda i: (i, 0, 0)),
            pl.BlockSpec((1, NT_, 128), lambda i: (0, 0, 0)),
            pl.BlockSpec((CBLK, BINS_), lambda i: (i, 0)),
            pl.BlockSpec((C_, BINS_), lambda i: (0, 0)),
        ],
        out_specs=pl.BlockSpec((1, 1), lambda i: (0, 0)),
        out_shape=jax.ShapeDtypeStruct((1, 1), jnp.float32),
        scratch_shapes=[
            pltpu.VMEM((8, BINS_), jnp.float32),
            pltpu.SMEM((8,), jnp.float32),
        ],
    )(inp3, mask3, style_his, fine)

    return out[0, 0]
